# baseline (speedup 1.0000x reference)
import functools

import jax
import jax.numpy as jnp
from jax import lax
from jax.experimental import pallas as pl
from jax.experimental.pallas import tpu as pltpu
from jax.experimental.pallas import tpu_sc as plsc

NC = 2
NS = 16
NW = NC * NS
LANES = 16

DEPTH = 6
H = 128
HG = H // LANES


def _sc_mesh():
    return plsc.VectorSubcoreMesh(core_axis_name="c", subcore_axis_name="s")


def _ds8(off, size):
    return pl.ds(pl.multiple_of(off, 8), size)


CA = 4
MAX_NB = 32
CHUNK_IDX = CA * MAX_NB


NBUF = 4


def _seg_sum_kernel(n_atoms_pad):
    per_tile_chunks = n_atoms_pad // (CA * NW)
    per_tile_atoms = per_tile_chunks * CA
    per_tile_idx = per_tile_atoms * MAX_NB
    n_rounds = (per_tile_chunks + NBUF - 1) // NBUF

    @functools.partial(
        pl.kernel,
        out_type=jax.ShapeDtypeStruct((n_atoms_pad, H), jnp.float32),
        mesh=_sc_mesh(),
        scratch_types=[
            pltpu.VMEM((per_tile_chunks, CHUNK_IDX), jnp.int32),
        ] + [pltpu.VMEM((CHUNK_IDX, H), jnp.float32)] * NBUF + [
            pltpu.VMEM((per_tile_atoms, H), jnp.float32),
        ] + [pltpu.SemaphoreType.DMA] * NBUF,
    )
    def k(m_hbm, a2b_hbm, out_hbm, idx_v, *bufs):
        rows = bufs[:NBUF]
        out_v = bufs[NBUF]
        sems = bufs[NBUF + 1:]
        wid = lax.axis_index("s") * NC + lax.axis_index("c")
        pltpu.sync_copy(
            a2b_hbm.at[_ds8(wid * per_tile_chunks, per_tile_chunks)], idx_v)

        def start(j, b):
            pltpu.async_copy(m_hbm.at[idx_v.at[j]], rows[b], sems[b])

        def wait(b):
            pltpu.make_async_copy(m_hbm.at[idx_v.at[0]],
                                  rows[b], sems[b]).wait()

        for b in range(NBUF):
            start(b, b)

        @pl.loop(0, n_rounds)
        def _(p):
            for b in range(NBUF):
                j = p * NBUF + b

                @pl.when(j < per_tile_chunks)
                def _(j=j, b=b):
                    wait(b)
                    rbuf = rows[b]
                    for a in range(CA):
                        orow = j * CA + a
                        for g in range(HG):
                            sl = pl.ds(g * LANES, LANES)
                            out_v[orow, sl] = rbuf[a * MAX_NB, sl]

                        @pl.loop(1, MAX_NB)
                        def _(kk, a=a, orow=orow):
                            row = a * MAX_NB + kk
                            for g in range(HG):
                                sl = pl.ds(g * LANES, LANES)
                                plsc.addupdate(out_v.at[orow, sl],
                                               rbuf[row, sl])

                    @pl.when(j + NBUF < per_tile_chunks)
                    def _():
                        start(j + NBUF, b)

        pltpu.sync_copy(out_v,
                        out_hbm.at[_ds8(wid * per_tile_atoms,
                                        per_tile_atoms)])

    return k


CB = 80


def _gather_sub_kernel(n_bonds):
    per_tile = n_bonds // NW
    n_chunks = per_tile // CB
    n_pairs = (n_chunks + 1) // 2

    @functools.partial(
        pl.kernel,
        out_type=jax.ShapeDtypeStruct((n_bonds, H), jnp.float32),
        mesh=_sc_mesh(),
        scratch_types=[
            pltpu.VMEM((n_chunks, CB), jnp.int32),
            pltpu.VMEM((n_chunks, CB), jnp.int32),
            pltpu.VMEM((CB, H), jnp.float32),
            pltpu.VMEM((CB, H), jnp.float32),
            pltpu.VMEM((CB, H), jnp.float32),
            pltpu.VMEM((CB, H), jnp.float32),
            pltpu.SemaphoreType.DMA,
            pltpu.SemaphoreType.DMA,
            pltpu.SemaphoreType.DMA,
            pltpu.SemaphoreType.DMA,
            pltpu.SemaphoreType.DMA,
            pltpu.SemaphoreType.DMA,
        ],
    )
    def k(a_hbm, m_hbm, b2a_hbm, b2revb_hbm, out_hbm,
          idx1_v, idx2_v, ga0, ga1, gm0, gm1, sa0, sa1, sm0, sm1, sw0, sw1):
        wid = lax.axis_index("s") * NC + lax.axis_index("c")
        base = wid * per_tile
        pltpu.sync_copy(b2a_hbm.at[wid], idx1_v)
        pltpu.sync_copy(b2revb_hbm.at[wid], idx2_v)
        ga = (ga0, ga1)
        gm = (gm0, gm1)
        sa = (sa0, sa1)
        sm = (sm0, sm1)
        sw = (sw0, sw1)

        def start(j, b):
            pltpu.async_copy(a_hbm.at[idx1_v.at[j]], ga[b], sa[b])
            pltpu.async_copy(m_hbm.at[idx2_v.at[j]], gm[b], sm[b])

        def wait_gathers(b):
            pltpu.make_async_copy(a_hbm.at[idx1_v.at[0]],
                                  ga[b], sa[b]).wait()
            pltpu.make_async_copy(m_hbm.at[idx2_v.at[0]],
                                  gm[b], sm[b]).wait()

        def wait_write(b):
            pltpu.make_async_copy(ga[b], out_hbm.at[_ds8(base, CB)],
                                  sw[b]).wait()

        start(0, 0)

        @pl.loop(0, n_pairs)
        def _(p):
            for half in range(2):
                j = p * 2 + half

                @pl.when(j < n_chunks)
                def _(j=j, half=half):
                    @pl.when(j + 1 < n_chunks)
                    def _():
                        @pl.when(j >= 1)
                        def _():
                            wait_write(1 - half)
                        start(j + 1, 1 - half)

                    wait_gathers(half)
                    gab = ga[half]
                    gmb = gm[half]

                    @pl.loop(0, CB)
                    def _(r):
                        for g in range(HG):
                            sl = pl.ds(g * LANES, LANES)
                            gab[r, sl] = gab[r, sl] - gmb[r, sl]

                    pltpu.async_copy(gab,
                                     out_hbm.at[_ds8(base + j * CB, CB)],
                                     sw[half])

        wait_write(0)
        wait_write(1)

    return k


def _k1_call(f_bonds, w_i):
    n_bonds, fdim = f_bonds.shape
    br = 2560
    grid = (n_bonds // br,)

    def body(fb_ref, w_ref, inp_ref, m_ref):
        x = jnp.dot(fb_ref[...], w_ref[...],
                    preferred_element_type=jnp.float32)
        inp_ref[...] = x
        m_ref[...] = jnp.maximum(x, 0.0)

    return pl.pallas_call(
        body,
        grid=grid,
        in_specs=[
            pl.BlockSpec((br, fdim), lambda i: (i, 0)),
            pl.BlockSpec((fdim, H), lambda i: (0, 0)),
        ],
        out_specs=[
            pl.BlockSpec((br, H), lambda i: (i, 0)),
            pl.BlockSpec((br, H), lambda i: (i, 0)),
        ],
        out_shape=[
            jax.ShapeDtypeStruct((n_bonds, H), jnp.float32),
            jax.ShapeDtypeStruct((n_bonds, H), jnp.float32),
        ],
    )(f_bonds, w_i)


def _k3_call(t, inp, w_h):
    n_bonds = t.shape[0]
    br = 2560
    grid = (n_bonds // br,)

    def body(t_ref, i_ref, w_ref, m_ref):
        x = jnp.dot(t_ref[...], w_ref[...],
                    preferred_element_type=jnp.float32)
        m_ref[...] = jnp.maximum(i_ref[...] + x, 0.0)

    return pl.pallas_call(
        body,
        grid=grid,
        in_specs=[
            pl.BlockSpec((br, H), lambda i: (i, 0)),
            pl.BlockSpec((br, H), lambda i: (i, 0)),
            pl.BlockSpec((H, H), lambda i: (0, 0)),
        ],
        out_specs=pl.BlockSpec((br, H), lambda i: (i, 0)),
        out_shape=jax.ShapeDtypeStruct((n_bonds, H), jnp.float32),
    )(t, inp, w_h)


def _k4_call(f_atoms, a_msg, w_oa, w_om, b_o, n_mols, mol_size):
    n_atoms, fdim = f_atoms.shape
    mpb = 4
    apb = mpb * mol_size
    grid = (n_mols // mpb,)

    def body(fa_ref, am_ref, woa_ref, wom_ref, b_ref, out_ref):
        h = jnp.dot(fa_ref[...], woa_ref[...],
                    preferred_element_type=jnp.float32)
        h = h + jnp.dot(am_ref[...], wom_ref[...],
                        preferred_element_type=jnp.float32)
        h = jnp.maximum(h + b_ref[...], 0.0)
        inv = 1.0 / mol_size
        for m in range(mpb):
            s = jnp.sum(h[m * mol_size:(m + 1) * mol_size, :], axis=0) * inv
            out_ref[0, m, :] = s

    out = pl.pallas_call(
        body,
        grid=grid,
        in_specs=[
            pl.BlockSpec((apb, fdim), lambda i: (i, 0)),
            pl.BlockSpec((apb, H), lambda i: (i, 0)),
            pl.BlockSpec((fdim, H), lambda i: (0, 0)),
            pl.BlockSpec((H, H), lambda i: (0, 0)),
            pl.BlockSpec((1, H), lambda i: (0, 0)),
        ],
        out_specs=pl.BlockSpec((1, mpb, H), lambda i: (i, 0, 0)),
        out_shape=jax.ShapeDtypeStruct((n_mols // mpb, mpb, H), jnp.float32),
    )(f_atoms, a_msg, w_oa, w_om, b_o)
    return out.reshape(n_mols, H)


def kernel(f_atoms, f_bonds, a2b, b2a, b2revb, a_scope, W_i, W_h, W_o, b_o):
    n_atoms, fdim_a = f_atoms.shape
    n_bonds = f_bonds.shape[0]
    n_mols = a_scope.shape[0]
    mol_size = n_atoms // n_mols

    atoms_per_tile = -(-n_atoms // (8 * NW)) * 8
    n_atoms_pad = atoms_per_tile * NW
    a2b_pad = jnp.pad(a2b, ((0, n_atoms_pad - n_atoms), (0, 0)))
    a2b_flat = a2b_pad.reshape(-1, CA * MAX_NB)
    b2a_2d = b2a.reshape(NW, -1, CB)
    b2revb_2d = b2revb.reshape(NW, -1, CB)

    seg_sum = _seg_sum_kernel(n_atoms_pad)
    gather_sub = _gather_sub_kernel(n_bonds)

    inp, msg = _k1_call(f_bonds, W_i)
    for _ in range(DEPTH - 1):
        a_msg = seg_sum(msg, a2b_flat)
        t = gather_sub(a_msg, msg, b2a_2d, b2revb_2d)
        msg = _k3_call(t, inp, W_h)

    a_msg = seg_sum(msg, a2b_flat)
    w_oa = W_o[:fdim_a]
    w_om = W_o[fdim_a:]
    return _k4_call(f_atoms, a_msg, w_oa, w_om, b_o.reshape(1, H),
                    n_mols, mol_size)


# speedup vs baseline: 1.2391x; 1.2391x over previous
import functools

import jax
import jax.numpy as jnp
from jax import lax
from jax.experimental import pallas as pl
from jax.experimental.pallas import tpu as pltpu
from jax.experimental.pallas import tpu_sc as plsc

NC = 2
NS = 16
NW = NC * NS
LANES = 16

DEPTH = 6
H = 128
HG = H // LANES


def _sc_mesh():
    return plsc.VectorSubcoreMesh(core_axis_name="c", subcore_axis_name="s")


CA = 4
MAX_NB = 32
CHUNK_IDX = CA * MAX_NB


def _seg_sum_kernel(n_atoms):
    n_chunks = n_atoms // CA
    n_iters = (n_chunks + NW - 1) // NW

    @functools.partial(
        pl.kernel,
        out_type=jax.ShapeDtypeStruct((n_atoms, H), jnp.float32),
        mesh=_sc_mesh(),
        scratch_types=[
            pltpu.VMEM((CHUNK_IDX,), jnp.int32),
            pltpu.VMEM((CHUNK_IDX, H), jnp.float32),
            pltpu.VMEM((CA, H), jnp.float32),
            pltpu.SemaphoreType.DMA,
        ],
    )
    def k(m_hbm, a2b_hbm, out_hbm, idx_v, rows_v, out_v, sem):
        wid = lax.axis_index("s") * NC + lax.axis_index("c")

        @pl.loop(0, n_iters)
        def _(it):
            c = it * NW + wid

            @pl.when(c < n_chunks)
            def _():
                pltpu.sync_copy(a2b_hbm.at[pl.ds(c * CHUNK_IDX, CHUNK_IDX)],
                                idx_v)
                pltpu.async_copy(m_hbm.at[idx_v], rows_v, sem).wait()
                for a in range(CA):
                    def body(kk, accs, a=a):
                        row = a * MAX_NB + kk
                        return tuple(
                            accs[g] + rows_v[row, pl.ds(g * LANES, LANES)]
                            for g in range(HG))
                    accs = lax.fori_loop(
                        0, MAX_NB, body,
                        tuple(jnp.zeros((LANES,), jnp.float32)
                              for _ in range(HG)))
                    for g in range(HG):
                        out_v[a, pl.ds(g * LANES, LANES)] = accs[g]
                pltpu.sync_copy(out_v, out_hbm.at[pl.ds(c * CA, CA)])

    return k


CB = 128


def _gather_sub_kernel(n_bonds):
    n_chunks = n_bonds // CB
    n_iters = (n_chunks + NW - 1) // NW

    @functools.partial(
        pl.kernel,
        out_type=jax.ShapeDtypeStruct((n_bonds, H), jnp.float32),
        mesh=_sc_mesh(),
        scratch_types=[
            pltpu.VMEM((CB,), jnp.int32),
            pltpu.VMEM((CB,), jnp.int32),
            pltpu.VMEM((CB, H), jnp.float32),
            pltpu.VMEM((CB, H), jnp.float32),
            pltpu.SemaphoreType.DMA,
            pltpu.SemaphoreType.DMA,
        ],
    )
    def k(a_hbm, m_hbm, b2a_hbm, b2revb_hbm, out_hbm,
          idx1_v, idx2_v, ga_v, gm_v, sem1, sem2):
        wid = lax.axis_index("s") * NC + lax.axis_index("c")

        @pl.loop(0, n_iters)
        def _(it):
            c = it * NW + wid

            @pl.when(c < n_chunks)
            def _():
                base = c * CB
                pltpu.sync_copy(b2a_hbm.at[pl.ds(base, CB)], idx1_v)
                pltpu.sync_copy(b2revb_hbm.at[pl.ds(base, CB)], idx2_v)
                cp1 = pltpu.async_copy(a_hbm.at[idx1_v], ga_v, sem1)
                cp2 = pltpu.async_copy(m_hbm.at[idx2_v], gm_v, sem2)
                cp1.wait()
                cp2.wait()

                @pl.loop(0, CB)
                def _(r):
                    for g in range(HG):
                        sl = pl.ds(g * LANES, LANES)
                        ga_v[r, sl] = ga_v[r, sl] - gm_v[r, sl]

                pltpu.sync_copy(ga_v, out_hbm.at[pl.ds(base, CB)])

    return k


def _k1_call(f_bonds, w_i):
    n_bonds, fdim = f_bonds.shape
    br = 2560
    grid = (n_bonds // br,)

    def body(fb_ref, w_ref, inp_ref, m_ref):
        x = jnp.dot(fb_ref[...], w_ref[...],
                    preferred_element_type=jnp.float32)
        inp_ref[...] = x
        m_ref[...] = jnp.maximum(x, 0.0)

    return pl.pallas_call(
        body,
        grid=grid,
        in_specs=[
            pl.BlockSpec((br, fdim), lambda i: (i, 0)),
            pl.BlockSpec((fdim, H), lambda i: (0, 0)),
        ],
        out_specs=[
            pl.BlockSpec((br, H), lambda i: (i, 0)),
            pl.BlockSpec((br, H), lambda i: (i, 0)),
        ],
        out_shape=[
            jax.ShapeDtypeStruct((n_bonds, H), jnp.float32),
            jax.ShapeDtypeStruct((n_bonds, H), jnp.float32),
        ],
    )(f_bonds, w_i)


def _k3_call(t, inp, w_h):
    n_bonds = t.shape[0]
    br = 2560
    grid = (n_bonds // br,)

    def body(t_ref, i_ref, w_ref, m_ref):
        x = jnp.dot(t_ref[...], w_ref[...],
                    preferred_element_type=jnp.float32)
        m_ref[...] = jnp.maximum(i_ref[...] + x, 0.0)

    return pl.pallas_call(
        body,
        grid=grid,
        in_specs=[
            pl.BlockSpec((br, H), lambda i: (i, 0)),
            pl.BlockSpec((br, H), lambda i: (i, 0)),
            pl.BlockSpec((H, H), lambda i: (0, 0)),
        ],
        out_specs=pl.BlockSpec((br, H), lambda i: (i, 0)),
        out_shape=jax.ShapeDtypeStruct((n_bonds, H), jnp.float32),
    )(t, inp, w_h)


def _k4_call(f_atoms, a_msg, w_oa, w_om, b_o, n_mols, mol_size):
    n_atoms, fdim = f_atoms.shape
    mpb = 4
    apb = mpb * mol_size
    grid = (n_mols // mpb,)

    def body(fa_ref, am_ref, woa_ref, wom_ref, b_ref, out_ref):
        h = jnp.dot(fa_ref[...], woa_ref[...],
                    preferred_element_type=jnp.float32)
        h = h + jnp.dot(am_ref[...], wom_ref[...],
                        preferred_element_type=jnp.float32)
        h = jnp.maximum(h + b_ref[...], 0.0)
        inv = 1.0 / mol_size
        for m in range(mpb):
            s = jnp.sum(h[m * mol_size:(m + 1) * mol_size, :], axis=0) * inv
            out_ref[0, m, :] = s

    out = pl.pallas_call(
        body,
        grid=grid,
        in_specs=[
            pl.BlockSpec((apb, fdim), lambda i: (i, 0)),
            pl.BlockSpec((apb, H), lambda i: (i, 0)),
            pl.BlockSpec((fdim, H), lambda i: (0, 0)),
            pl.BlockSpec((H, H), lambda i: (0, 0)),
            pl.BlockSpec((1, H), lambda i: (0, 0)),
        ],
        out_specs=pl.BlockSpec((1, mpb, H), lambda i: (i, 0, 0)),
        out_shape=jax.ShapeDtypeStruct((n_mols // mpb, mpb, H), jnp.float32),
    )(f_atoms, a_msg, w_oa, w_om, b_o)
    return out.reshape(n_mols, H)


def kernel(f_atoms, f_bonds, a2b, b2a, b2revb, a_scope, W_i, W_h, W_o, b_o):
    n_atoms, fdim_a = f_atoms.shape
    n_bonds = f_bonds.shape[0]
    n_mols = a_scope.shape[0]
    mol_size = n_atoms // n_mols

    a2b_flat = a2b.reshape(-1)
    seg_sum = _seg_sum_kernel(n_atoms)
    gather_sub = _gather_sub_kernel(n_bonds)

    inp, msg = _k1_call(f_bonds, W_i)
    for _ in range(DEPTH - 1):
        a_msg = seg_sum(msg, a2b_flat)
        t = gather_sub(a_msg, msg, b2a, b2revb)
        msg = _k3_call(t, inp, W_h)

    a_msg = seg_sum(msg, a2b_flat)
    w_oa = W_o[:fdim_a]
    w_om = W_o[fdim_a:]
    return _k4_call(f_atoms, a_msg, w_oa, w_om, b_o.reshape(1, H),
                    n_mols, mol_size)


# speedup vs baseline: 1.5272x; 1.2325x over previous
import functools

import jax
import jax.numpy as jnp
from jax import lax
from jax.experimental import pallas as pl
from jax.experimental.pallas import tpu as pltpu
from jax.experimental.pallas import tpu_sc as plsc

NC = 2
NS = 16
NW = NC * NS
LANES = 16

DEPTH = 6
H = 128
HG = H // LANES


def _sc_mesh():
    return plsc.VectorSubcoreMesh(core_axis_name="c", subcore_axis_name="s")


CA = 4
MAX_NB = 32
CHUNK_IDX = CA * MAX_NB


def _seg_sum_kernel(n_atoms):
    n_chunks = n_atoms // CA
    n_iters = (n_chunks + NW - 1) // NW

    @functools.partial(
        pl.kernel,
        out_type=jax.ShapeDtypeStruct((n_atoms, H), jnp.float32),
        mesh=_sc_mesh(),
        scratch_types=[
            pltpu.VMEM((CHUNK_IDX,), jnp.int32),
            pltpu.VMEM((CHUNK_IDX, H), jnp.float32),
            pltpu.VMEM((CA, H), jnp.float32),
            pltpu.SemaphoreType.DMA,
            pltpu.SemaphoreType.DMA,
            pltpu.SemaphoreType.DMA,
        ],
    )
    def k(m_hbm, a2b_hbm, out_hbm, idx_v, rows_v, out_v,
          sem_i, sem_g, sem_w):
        wid = lax.axis_index("s") * NC + lax.axis_index("c")

        def start_idx(c):
            pltpu.async_copy(a2b_hbm.at[pl.ds(c * CHUNK_IDX, CHUNK_IDX)],
                             idx_v, sem_i)

        def wait_idx():
            pltpu.make_async_copy(a2b_hbm.at[pl.ds(0, CHUNK_IDX)],
                                  idx_v, sem_i).wait()

        def wait_write():
            pltpu.make_async_copy(out_v, out_hbm.at[pl.ds(0, CA)],
                                  sem_w).wait()

        start_idx(wid)

        @pl.loop(0, n_iters)
        def _(it):
            c = it * NW + wid

            @pl.when(c < n_chunks)
            def _():
                wait_idx()
                pltpu.async_copy(m_hbm.at[idx_v], rows_v, sem_g).wait()

                @pl.when(c + NW < n_chunks)
                def _():
                    start_idx(c + NW)

                @pl.when(it > 0)
                def _():
                    wait_write()
                for a in range(CA):
                    def body(kk, accs, a=a):
                        row = a * MAX_NB + kk
                        return tuple(
                            accs[g] + rows_v[row, pl.ds(g * LANES, LANES)]
                            for g in range(HG))
                    accs = lax.fori_loop(
                        0, MAX_NB, body,
                        tuple(jnp.zeros((LANES,), jnp.float32)
                              for _ in range(HG)))
                    for g in range(HG):
                        out_v[a, pl.ds(g * LANES, LANES)] = accs[g]
                pltpu.async_copy(out_v, out_hbm.at[pl.ds(c * CA, CA)],
                                 sem_w)

        wait_write()

    return k


CB = 128


def _gather_sub_kernel(n_bonds):
    n_chunks = n_bonds // CB
    n_iters = (n_chunks + NW - 1) // NW

    @functools.partial(
        pl.kernel,
        out_type=jax.ShapeDtypeStruct((n_bonds, H), jnp.float32),
        mesh=_sc_mesh(),
        scratch_types=[
            pltpu.VMEM((CB,), jnp.int32),
            pltpu.VMEM((CB,), jnp.int32),
            pltpu.VMEM((CB, H), jnp.float32),
            pltpu.VMEM((CB, H), jnp.float32),
            pltpu.VMEM((CB, H), jnp.float32),
            pltpu.SemaphoreType.DMA,
            pltpu.SemaphoreType.DMA,
            pltpu.SemaphoreType.DMA,
            pltpu.SemaphoreType.DMA,
        ],
    )
    def k(a_hbm, m_hbm, b2a_hbm, b2revb_hbm, out_hbm,
          idx1_v, idx2_v, ga_v, gm_v, to_v, sem1, sem2, sem_i, sem_w):
        wid = lax.axis_index("s") * NC + lax.axis_index("c")

        def start_idx(c):
            pltpu.async_copy(b2a_hbm.at[pl.ds(c * CB, CB)], idx1_v, sem_i)
            pltpu.async_copy(b2revb_hbm.at[pl.ds(c * CB, CB)], idx2_v,
                             sem_i)

        def wait_idx():
            pltpu.make_async_copy(b2a_hbm.at[pl.ds(0, CB)], idx1_v,
                                  sem_i).wait()
            pltpu.make_async_copy(b2revb_hbm.at[pl.ds(0, CB)], idx2_v,
                                  sem_i).wait()

        def wait_write():
            pltpu.make_async_copy(to_v, out_hbm.at[pl.ds(0, CB)],
                                  sem_w).wait()

        start_idx(wid)

        @pl.loop(0, n_iters)
        def _(it):
            c = it * NW + wid

            @pl.when(c < n_chunks)
            def _():
                wait_idx()
                cp1 = pltpu.async_copy(a_hbm.at[idx1_v], ga_v, sem1)
                cp2 = pltpu.async_copy(m_hbm.at[idx2_v], gm_v, sem2)
                cp1.wait()
                cp2.wait()

                @pl.when(c + NW < n_chunks)
                def _():
                    start_idx(c + NW)

                @pl.when(it > 0)
                def _():
                    wait_write()

                @pl.loop(0, CB)
                def _(r):
                    for g in range(HG):
                        sl = pl.ds(g * LANES, LANES)
                        to_v[r, sl] = ga_v[r, sl] - gm_v[r, sl]

                pltpu.async_copy(to_v, out_hbm.at[pl.ds(c * CB, CB)],
                                 sem_w)

        wait_write()

    return k


def _k1_call(f_bonds, w_i):
    n_bonds, fdim = f_bonds.shape
    br = 2560
    grid = (n_bonds // br,)

    def body(fb_ref, w_ref, inp_ref, m_ref):
        x = jnp.dot(fb_ref[...], w_ref[...],
                    preferred_element_type=jnp.float32)
        inp_ref[...] = x
        m_ref[...] = jnp.maximum(x, 0.0)

    return pl.pallas_call(
        body,
        grid=grid,
        in_specs=[
            pl.BlockSpec((br, fdim), lambda i: (i, 0)),
            pl.BlockSpec((fdim, H), lambda i: (0, 0)),
        ],
        out_specs=[
            pl.BlockSpec((br, H), lambda i: (i, 0)),
            pl.BlockSpec((br, H), lambda i: (i, 0)),
        ],
        out_shape=[
            jax.ShapeDtypeStruct((n_bonds, H), jnp.float32),
            jax.ShapeDtypeStruct((n_bonds, H), jnp.float32),
        ],
    )(f_bonds, w_i)


def _k3_call(t, inp, w_h):
    n_bonds = t.shape[0]
    br = 2560
    grid = (n_bonds // br,)

    def body(t_ref, i_ref, w_ref, m_ref):
        x = jnp.dot(t_ref[...], w_ref[...],
                    preferred_element_type=jnp.float32)
        m_ref[...] = jnp.maximum(i_ref[...] + x, 0.0)

    return pl.pallas_call(
        body,
        grid=grid,
        in_specs=[
            pl.BlockSpec((br, H), lambda i: (i, 0)),
            pl.BlockSpec((br, H), lambda i: (i, 0)),
            pl.BlockSpec((H, H), lambda i: (0, 0)),
        ],
        out_specs=pl.BlockSpec((br, H), lambda i: (i, 0)),
        out_shape=jax.ShapeDtypeStruct((n_bonds, H), jnp.float32),
    )(t, inp, w_h)


def _k4_call(f_atoms, a_msg, w_oa, w_om, b_o, n_mols, mol_size):
    n_atoms, fdim = f_atoms.shape
    mpb = 4
    apb = mpb * mol_size
    grid = (n_mols // mpb,)

    def body(fa_ref, am_ref, woa_ref, wom_ref, b_ref, out_ref):
        h = jnp.dot(fa_ref[...], woa_ref[...],
                    preferred_element_type=jnp.float32)
        h = h + jnp.dot(am_ref[...], wom_ref[...],
                        preferred_element_type=jnp.float32)
        h = jnp.maximum(h + b_ref[...], 0.0)
        inv = 1.0 / mol_size
        for m in range(mpb):
            s = jnp.sum(h[m * mol_size:(m + 1) * mol_size, :], axis=0) * inv
            out_ref[0, m, :] = s

    out = pl.pallas_call(
        body,
        grid=grid,
        in_specs=[
            pl.BlockSpec((apb, fdim), lambda i: (i, 0)),
            pl.BlockSpec((apb, H), lambda i: (i, 0)),
            pl.BlockSpec((fdim, H), lambda i: (0, 0)),
            pl.BlockSpec((H, H), lambda i: (0, 0)),
            pl.BlockSpec((1, H), lambda i: (0, 0)),
        ],
        out_specs=pl.BlockSpec((1, mpb, H), lambda i: (i, 0, 0)),
        out_shape=jax.ShapeDtypeStruct((n_mols // mpb, mpb, H), jnp.float32),
    )(f_atoms, a_msg, w_oa, w_om, b_o)
    return out.reshape(n_mols, H)


def kernel(f_atoms, f_bonds, a2b, b2a, b2revb, a_scope, W_i, W_h, W_o, b_o):
    n_atoms, fdim_a = f_atoms.shape
    n_bonds = f_bonds.shape[0]
    n_mols = a_scope.shape[0]
    mol_size = n_atoms // n_mols

    a2b_flat = a2b.reshape(-1)
    seg_sum = _seg_sum_kernel(n_atoms)
    gather_sub = _gather_sub_kernel(n_bonds)

    inp, msg = _k1_call(f_bonds, W_i)
    for _ in range(DEPTH - 1):
        a_msg = seg_sum(msg, a2b_flat)
        t = gather_sub(a_msg, msg, b2a, b2revb)
        msg = _k3_call(t, inp, W_h)

    a_msg = seg_sum(msg, a2b_flat)
    w_oa = W_o[:fdim_a]
    w_om = W_o[fdim_a:]
    return _k4_call(f_atoms, a_msg, w_oa, w_om, b_o.reshape(1, H),
                    n_mols, mol_size)


# speedup vs baseline: 1.6507x; 1.0809x over previous
import functools

import jax
import jax.numpy as jnp
from jax import lax
from jax.experimental import pallas as pl
from jax.experimental.pallas import tpu as pltpu
from jax.experimental.pallas import tpu_sc as plsc

NC = 2
NS = 16
NW = NC * NS
LANES = 16

DEPTH = 6
H = 128
HG = H // LANES


def _sc_mesh():
    return plsc.VectorSubcoreMesh(core_axis_name="c", subcore_axis_name="s")


CA = 4
MAX_NB = 32
CHUNK_IDX = CA * MAX_NB


def _seg_sum_kernel(n_atoms):
    n_chunks = n_atoms // CA
    n_iters = (n_chunks + NW - 1) // NW

    @functools.partial(
        pl.kernel,
        out_type=jax.ShapeDtypeStruct((n_atoms, H), jnp.float32),
        mesh=_sc_mesh(),
        scratch_types=[
            pltpu.VMEM((CHUNK_IDX,), jnp.int32),
            pltpu.VMEM((CHUNK_IDX,), jnp.int32),
            pltpu.VMEM((CHUNK_IDX, H), jnp.float32),
            pltpu.VMEM((CHUNK_IDX, H), jnp.float32),
            pltpu.VMEM((CA, H), jnp.float32),
            pltpu.SemaphoreType.DMA,
            pltpu.SemaphoreType.DMA,
            pltpu.SemaphoreType.DMA,
            pltpu.SemaphoreType.DMA,
            pltpu.SemaphoreType.DMA,
        ],
    )
    def k(m_hbm, a2b_hbm, out_hbm, idx0, idx1, rows0, rows1, out_v,
          si0, si1, sg0, sg1, sem_w):
        wid = lax.axis_index("s") * NC + lax.axis_index("c")
        idx = (idx0, idx1)
        rows = (rows0, rows1)
        si = (si0, si1)
        sg = (sg0, sg1)

        def start_idx(c, b):
            pltpu.async_copy(a2b_hbm.at[pl.ds(c * CHUNK_IDX, CHUNK_IDX)],
                             idx[b], si[b])

        def wait_idx(b):
            pltpu.make_async_copy(a2b_hbm.at[pl.ds(0, CHUNK_IDX)],
                                  idx[b], si[b]).wait()

        def start_gather(b):
            pltpu.async_copy(m_hbm.at[idx[b]], rows[b], sg[b])

        def wait_gather(b):
            pltpu.make_async_copy(m_hbm.at[idx[b]], rows[b], sg[b]).wait()

        def wait_write():
            pltpu.make_async_copy(out_v, out_hbm.at[pl.ds(0, CA)],
                                  sem_w).wait()

        start_idx(wid, 0)
        start_idx(wid + NW, 1)
        wait_idx(0)
        start_gather(0)
        n_pairs = (n_iters + 1) // 2

        @pl.loop(0, n_pairs)
        def _(p):
            for half in range(2):
                it = p * 2 + half
                c = it * NW + wid

                @pl.when(c < n_chunks)
                def _(it=it, c=c, half=half):
                    wait_gather(half)

                    @pl.when(c + 2 * NW < n_chunks)
                    def _():
                        start_idx(c + 2 * NW, half)

                    @pl.when(c + NW < n_chunks)
                    def _():
                        wait_idx(1 - half)
                        start_gather(1 - half)

                    @pl.when(it > 0)
                    def _():
                        wait_write()
                    rbuf = rows[half]
                    for a in range(CA):
                        def body(kk, accs, a=a):
                            row = a * MAX_NB + kk
                            return tuple(
                                accs[g] + rbuf[row, pl.ds(g * LANES, LANES)]
                                for g in range(HG))
                        accs = lax.fori_loop(
                            0, MAX_NB, body,
                            tuple(jnp.zeros((LANES,), jnp.float32)
                                  for _ in range(HG)))
                        for g in range(HG):
                            out_v[a, pl.ds(g * LANES, LANES)] = accs[g]
                    pltpu.async_copy(out_v, out_hbm.at[pl.ds(c * CA, CA)],
                                     sem_w)

        wait_write()

    return k


CB = 128


def _gather_sub_kernel(n_bonds):
    n_chunks = n_bonds // CB
    n_iters = (n_chunks + NW - 1) // NW

    @functools.partial(
        pl.kernel,
        out_type=jax.ShapeDtypeStruct((n_bonds, H), jnp.float32),
        mesh=_sc_mesh(),
        scratch_types=[
            pltpu.VMEM((CB,), jnp.int32),
            pltpu.VMEM((CB,), jnp.int32),
            pltpu.VMEM((CB, H), jnp.float32),
            pltpu.VMEM((CB, H), jnp.float32),
            pltpu.VMEM((CB, H), jnp.float32),
            pltpu.SemaphoreType.DMA,
            pltpu.SemaphoreType.DMA,
            pltpu.SemaphoreType.DMA,
            pltpu.SemaphoreType.DMA,
        ],
    )
    def k(a_hbm, m_hbm, b2a_hbm, b2revb_hbm, out_hbm,
          idx1_v, idx2_v, ga_v, gm_v, to_v, sem1, sem2, sem_i, sem_w):
        wid = lax.axis_index("s") * NC + lax.axis_index("c")

        def start_idx(c):
            pltpu.async_copy(b2a_hbm.at[pl.ds(c * CB, CB)], idx1_v, sem_i)
            pltpu.async_copy(b2revb_hbm.at[pl.ds(c * CB, CB)], idx2_v,
                             sem_i)

        def wait_idx():
            pltpu.make_async_copy(b2a_hbm.at[pl.ds(0, CB)], idx1_v,
                                  sem_i).wait()
            pltpu.make_async_copy(b2revb_hbm.at[pl.ds(0, CB)], idx2_v,
                                  sem_i).wait()

        def wait_write():
            pltpu.make_async_copy(to_v, out_hbm.at[pl.ds(0, CB)],
                                  sem_w).wait()

        start_idx(wid)

        @pl.loop(0, n_iters)
        def _(it):
            c = it * NW + wid

            @pl.when(c < n_chunks)
            def _():
                wait_idx()
                cp1 = pltpu.async_copy(a_hbm.at[idx1_v], ga_v, sem1)
                cp2 = pltpu.async_copy(m_hbm.at[idx2_v], gm_v, sem2)
                cp1.wait()
                cp2.wait()

                @pl.when(c + NW < n_chunks)
                def _():
                    start_idx(c + NW)

                @pl.when(it > 0)
                def _():
                    wait_write()

                @pl.loop(0, CB)
                def _(r):
                    for g in range(HG):
                        sl = pl.ds(g * LANES, LANES)
                        to_v[r, sl] = ga_v[r, sl] - gm_v[r, sl]

                pltpu.async_copy(to_v, out_hbm.at[pl.ds(c * CB, CB)],
                                 sem_w)

        wait_write()

    return k


def _k1_call(f_bonds, w_i):
    n_bonds, fdim = f_bonds.shape
    br = 2560
    grid = (n_bonds // br,)

    def body(fb_ref, w_ref, inp_ref, m_ref):
        x = jnp.dot(fb_ref[...], w_ref[...],
                    preferred_element_type=jnp.float32)
        inp_ref[...] = x
        m_ref[...] = jnp.maximum(x, 0.0)

    return pl.pallas_call(
        body,
        grid=grid,
        in_specs=[
            pl.BlockSpec((br, fdim), lambda i: (i, 0)),
            pl.BlockSpec((fdim, H), lambda i: (0, 0)),
        ],
        out_specs=[
            pl.BlockSpec((br, H), lambda i: (i, 0)),
            pl.BlockSpec((br, H), lambda i: (i, 0)),
        ],
        out_shape=[
            jax.ShapeDtypeStruct((n_bonds, H), jnp.float32),
            jax.ShapeDtypeStruct((n_bonds, H), jnp.float32),
        ],
    )(f_bonds, w_i)


def _k3_call(t, inp, w_h):
    n_bonds = t.shape[0]
    br = 2560
    grid = (n_bonds // br,)

    def body(t_ref, i_ref, w_ref, m_ref):
        x = jnp.dot(t_ref[...], w_ref[...],
                    preferred_element_type=jnp.float32)
        m_ref[...] = jnp.maximum(i_ref[...] + x, 0.0)

    return pl.pallas_call(
        body,
        grid=grid,
        in_specs=[
            pl.BlockSpec((br, H), lambda i: (i, 0)),
            pl.BlockSpec((br, H), lambda i: (i, 0)),
            pl.BlockSpec((H, H), lambda i: (0, 0)),
        ],
        out_specs=pl.BlockSpec((br, H), lambda i: (i, 0)),
        out_shape=jax.ShapeDtypeStruct((n_bonds, H), jnp.float32),
    )(t, inp, w_h)


def _k4_call(f_atoms, a_msg, w_oa, w_om, b_o, n_mols, mol_size):
    n_atoms, fdim = f_atoms.shape
    mpb = 4
    apb = mpb * mol_size
    grid = (n_mols // mpb,)

    def body(fa_ref, am_ref, woa_ref, wom_ref, b_ref, out_ref):
        h = jnp.dot(fa_ref[...], woa_ref[...],
                    preferred_element_type=jnp.float32)
        h = h + jnp.dot(am_ref[...], wom_ref[...],
                        preferred_element_type=jnp.float32)
        h = jnp.maximum(h + b_ref[...], 0.0)
        inv = 1.0 / mol_size
        for m in range(mpb):
            s = jnp.sum(h[m * mol_size:(m + 1) * mol_size, :], axis=0) * inv
            out_ref[0, m, :] = s

    out = pl.pallas_call(
        body,
        grid=grid,
        in_specs=[
            pl.BlockSpec((apb, fdim), lambda i: (i, 0)),
            pl.BlockSpec((apb, H), lambda i: (i, 0)),
            pl.BlockSpec((fdim, H), lambda i: (0, 0)),
            pl.BlockSpec((H, H), lambda i: (0, 0)),
            pl.BlockSpec((1, H), lambda i: (0, 0)),
        ],
        out_specs=pl.BlockSpec((1, mpb, H), lambda i: (i, 0, 0)),
        out_shape=jax.ShapeDtypeStruct((n_mols // mpb, mpb, H), jnp.float32),
    )(f_atoms, a_msg, w_oa, w_om, b_o)
    return out.reshape(n_mols, H)


def kernel(f_atoms, f_bonds, a2b, b2a, b2revb, a_scope, W_i, W_h, W_o, b_o):
    n_atoms, fdim_a = f_atoms.shape
    n_bonds = f_bonds.shape[0]
    n_mols = a_scope.shape[0]
    mol_size = n_atoms // n_mols

    a2b_flat = a2b.reshape(-1)
    seg_sum = _seg_sum_kernel(n_atoms)
    gather_sub = _gather_sub_kernel(n_bonds)

    inp, msg = _k1_call(f_bonds, W_i)
    for _ in range(DEPTH - 1):
        a_msg = seg_sum(msg, a2b_flat)
        t = gather_sub(a_msg, msg, b2a, b2revb)
        msg = _k3_call(t, inp, W_h)

    a_msg = seg_sum(msg, a2b_flat)
    w_oa = W_o[:fdim_a]
    w_om = W_o[fdim_a:]
    return _k4_call(f_atoms, a_msg, w_oa, w_om, b_o.reshape(1, H),
                    n_mols, mol_size)
